# baseline (speedup 1.0000x reference)
import jax
import jax.numpy as jnp
from jax import lax
from jax.experimental import pallas as pl
from jax.experimental.pallas import tpu as pltpu

_B, _Q, _C, _V = 4, 20000, 81, 117
_QB = 2560
_NQ = (_Q + _QB - 1) // _QB
_SUBJECT_CATEGORY_ID = 0


def _fused_body(obj_ref, verb_ref, sub_ref, objb_ref, a_ref, b_ref, c_ref,
                lab_ref, sco_ref, vs_ref, subo_ref, objo_ref):
    m = obj_ref[0]
    idx = jnp.zeros(m.shape, jnp.int32)
    for c in range(1, _C - 1):
        x = obj_ref[c]
        gt = x > m
        m = jnp.where(gt, x, m)
        idx = jnp.where(gt, jnp.int32(c), idx)
    m81 = jnp.maximum(m, obj_ref[_C - 1])
    s = jnp.exp(obj_ref[0] - m81)
    for c in range(1, _C):
        s = s + jnp.exp(obj_ref[c] - m81)
    score = jnp.exp(m - m81) / s

    lab_ref[...] = idx
    sco_ref[...] = score

    for v in range(_V):
        vb = verb_ref[v]
        vs_ref[v] = score / (1.0 + jnp.exp(-vb))

    a = a_ref[...]
    b2 = b_ref[...]
    c2 = c_ref[...]
    for sref, oref in ((sub_ref, subo_ref), (objb_ref, objo_ref)):
        x = sref[...]
        oref[...] = (x * a + jnp.roll(x, -2, axis=1) * b2
                     + jnp.roll(x, 2, axis=1) * c2)


def _postprocess(obj_t, verb_t, sub_t, objb_t, a, b, c):
    return pl.pallas_call(
        _fused_body,
        grid=(_NQ,),
        in_specs=[
            pl.BlockSpec((_C, _B, _QB), lambda q: (0, 0, q)),
            pl.BlockSpec((_V, _B, _QB), lambda q: (0, 0, q)),
            pl.BlockSpec((_B, 4, _QB), lambda q: (0, 0, q)),
            pl.BlockSpec((_B, 4, _QB), lambda q: (0, 0, q)),
            pl.BlockSpec((_B, 4, 1), lambda q: (0, 0, 0)),
            pl.BlockSpec((_B, 4, 1), lambda q: (0, 0, 0)),
            pl.BlockSpec((_B, 4, 1), lambda q: (0, 0, 0)),
        ],
        out_specs=(
            pl.BlockSpec((_B, _QB), lambda q: (0, q)),
            pl.BlockSpec((_B, _QB), lambda q: (0, q)),
            pl.BlockSpec((_V, _B, _QB), lambda q: (0, 0, q)),
            pl.BlockSpec((_B, 4, _QB), lambda q: (0, 0, q)),
            pl.BlockSpec((_B, 4, _QB), lambda q: (0, 0, q)),
        ),
        out_shape=(
            jax.ShapeDtypeStruct((_B, _Q), jnp.int32),
            jax.ShapeDtypeStruct((_B, _Q), jnp.float32),
            jax.ShapeDtypeStruct((_V, _B, _Q), jnp.float32),
            jax.ShapeDtypeStruct((_B, 4, _Q), jnp.float32),
            jax.ShapeDtypeStruct((_B, 4, _Q), jnp.float32),
        ),
    )(obj_t, verb_t, sub_t, objb_t, a, b, c)


def kernel(pred_obj_logits, pred_verb_logits, pred_sub_boxes, pred_obj_boxes, target_sizes):
    obj_t = jnp.transpose(pred_obj_logits, (2, 0, 1))
    verb_t = jnp.transpose(pred_verb_logits, (2, 0, 1))
    sub_t = jnp.transpose(pred_sub_boxes, (0, 2, 1))
    objb_t = jnp.transpose(pred_obj_boxes, (0, 2, 1))

    img_h = target_sizes[:, 0].astype(jnp.float32)
    img_w = target_sizes[:, 1].astype(jnp.float32)
    sf = jnp.stack([img_w, img_h, img_w, img_h], axis=1)
    a = (sf * jnp.array([1.0, 1.0, 0.5, 0.5])).reshape(_B, 4, 1)
    b = (sf * jnp.array([-0.5, -0.5, 0.0, 0.0])).reshape(_B, 4, 1)
    c = (sf * jnp.array([0.0, 0.0, 1.0, 1.0])).reshape(_B, 4, 1)

    obj_labels, obj_scores, vs_t, subo_t, objo_t = _postprocess(
        obj_t, verb_t, sub_t, objb_t, a, b, c)

    sl = jnp.full_like(obj_labels, _SUBJECT_CATEGORY_ID)
    labels = jnp.concatenate([sl, obj_labels], axis=1)
    vs = jnp.transpose(vs_t, (1, 2, 0))
    boxes_t = jnp.concatenate([subo_t, objo_t], axis=2)
    boxes = jnp.transpose(boxes_t, (0, 2, 1))

    ids = jnp.arange(2 * _Q)
    sub_ids = ids[:_Q]
    obj_ids = ids[_Q:]

    return (labels, boxes, vs, pred_verb_logits, sub_ids, obj_ids, obj_scores)
